# Initial kernel scaffold; baseline (speedup 1.0000x reference)
import jax
import jax.numpy as jnp
from jax.experimental import pallas as pl


def kernel(data, segment_ids):
    raise NotImplementedError("write your pallas kernel here")


# speedup vs baseline: 3.2894x; 3.2894x over previous
import jax
import jax.numpy as jnp
from jax import lax
from jax.experimental import pallas as pl
from jax.experimental.pallas import tpu as pltpu
from jax.experimental.pallas import tpu_sc as plsc

B, N, C = 4, 80000, 128
V = 10000
NC, NS, L = 2, 16, 16
BPC = B // NC
CHUNK = 128
NFULL = 39
TPT = CHUNK // NS
RCH = 40
NRND = (V // RCH) // NS
RTAIL0 = NRND * NS * RCH
NRT = (V - RTAIL0) // RCH


def _fill(ref, nrows, width, val):
    def body(r, _):
        for k in range(width // L):
            ref[r, pl.ds(k * L, L)] = jnp.full((L,), val, jnp.float32)
        return 0
    lax.fori_loop(0, nrows, body, 0)


def _touch(ref):
    x = ref[0, pl.ds(0, L)]
    ref[0, pl.ds(0, L)] = x


def _sc_body(data_hbm, ids_hbm, out_hbm, cstage_hbm, dbuf, idxbuf, onesb,
             cbuf, acc_sp):
    c = lax.axis_index("c")
    s = lax.axis_index("s")

    _fill(onesb, CHUNK, C, 1.0)

    def zero_acc():
        _fill(dbuf, RCH, C, 0.0)

        def zero_body(j, _):
            row0 = (j * NS + s) * RCH
            pltpu.sync_copy(dbuf.at[pl.ds(0, RCH)], acc_sp.at[pl.ds(row0, RCH)])
            return 0
        lax.fori_loop(0, NRND, zero_body, 0)
        for j2 in range(NRT):
            row0 = RTAIL0 + j2 * RCH
            pltpu.sync_copy(dbuf.at[pl.ds(0, RCH)], acc_sp.at[pl.ds(row0, RCH)])
        pltpu.sync_copy(dbuf.at[pl.ds(0, 8)], acc_sp.at[pl.ds(V, 8)])

    def scatter_pass(b, data_pass):
        def chunk_body(i, _):
            base = b * N + (i * NS + s) * CHUNK
            pltpu.sync_copy(ids_hbm.at[pl.ds(base, CHUNK)], idxbuf)
            xi = idxbuf[pl.ds(0, L)]
            idxbuf[pl.ds(0, L)] = xi
            if data_pass:
                pltpu.sync_copy(data_hbm.at[pl.ds(base, CHUNK)], dbuf)
                _touch(dbuf)
                pltpu.sync_copy(dbuf, acc_sp.at[idxbuf], add=True)
            else:
                pltpu.sync_copy(onesb, acc_sp.at[idxbuf], add=True)
            return 0
        lax.fori_loop(0, NFULL, chunk_body, 0)

        tbase = b * N + NFULL * NS * CHUNK
        pltpu.sync_copy(ids_hbm.at[pl.ds(tbase, CHUNK)], idxbuf)
        for k in range(CHUNK // L):
            ids_v = idxbuf[pl.ds(k * L, L)]
            p = k * L + lax.iota(jnp.int32, L)
            own = lax.shift_right_logical(p, 3) == s
            idxbuf[pl.ds(k * L, L)] = jnp.where(
                own, ids_v, jnp.full((L,), V, jnp.int32))
        if data_pass:
            pltpu.sync_copy(data_hbm.at[pl.ds(tbase, CHUNK)], dbuf)
            _touch(dbuf)
            pltpu.sync_copy(dbuf, acc_sp.at[idxbuf], add=True)
        else:
            pltpu.sync_copy(onesb, acc_sp.at[idxbuf], add=True)

    for bi in range(BPC):
        b = c * BPC + bi

        zero_acc()
        plsc.subcore_barrier()
        scatter_pass(b, data_pass=False)
        plsc.subcore_barrier()

        def stage_one(row0):
            pltpu.sync_copy(acc_sp.at[pl.ds(row0, RCH)], dbuf.at[pl.ds(0, RCH)])
            _touch(dbuf)
            pltpu.sync_copy(dbuf.at[pl.ds(0, RCH)],
                            cstage_hbm.at[pl.ds(c * V + row0, RCH)])

        def stage_body(j, _):
            stage_one((j * NS + s) * RCH)
            return 0
        lax.fori_loop(0, NRND, stage_body, 0)
        for j2 in range(NRT):
            stage_one(RTAIL0 + j2 * RCH)
        plsc.subcore_barrier()

        zero_acc()
        plsc.subcore_barrier()
        scatter_pass(b, data_pass=True)
        plsc.subcore_barrier()

        def div_one(row0):
            pltpu.sync_copy(acc_sp.at[pl.ds(row0, RCH)], dbuf.at[pl.ds(0, RCH)])
            pltpu.sync_copy(cstage_hbm.at[pl.ds(c * V + row0, RCH)], cbuf)

            def row_body(r, _):
                cnt = cbuf[r, pl.ds(0, L)]
                inv = 1.0 / jnp.maximum(cnt, 1.0)
                for k in range(C // L):
                    dbuf[r, pl.ds(k * L, L)] = dbuf[r, pl.ds(k * L, L)] * inv
                return 0
            lax.fori_loop(0, RCH, row_body, 0)

            pltpu.sync_copy(dbuf.at[pl.ds(0, RCH)],
                            out_hbm.at[pl.ds(b * V + row0, RCH)])

        def div_body(j, _):
            div_one((j * NS + s) * RCH)
            return 0
        lax.fori_loop(0, NRND, div_body, 0)
        for j2 in range(NRT):
            div_one(RTAIL0 + j2 * RCH)
        plsc.subcore_barrier()


@jax.jit
def _sc_pool(data, ids):
    mesh = plsc.VectorSubcoreMesh(core_axis_name="c", subcore_axis_name="s")
    f = pl.kernel(
        _sc_body,
        out_type=(
            jax.ShapeDtypeStruct((B * V, C), jnp.float32),
            jax.ShapeDtypeStruct((NC * V, C), jnp.float32),
        ),
        mesh=mesh,
        scratch_types=[
            pltpu.VMEM((CHUNK, C), jnp.float32),
            pltpu.VMEM((CHUNK,), jnp.int32),
            pltpu.VMEM((CHUNK, C), jnp.float32),
            pltpu.VMEM((RCH, C), jnp.float32),
            pltpu.VMEM_SHARED((V + 8, C), jnp.float32),
        ],
    )
    pooled, _ = f(data.reshape(B * N, C), ids.reshape(B * N))
    return pooled


def kernel(data, segment_ids):
    ids = segment_ids.astype(jnp.int32)
    pooled = _sc_pool(data, ids)
    maxes = ids[:, -1] + 1
    batch_offset = jnp.cumsum(maxes).astype(jnp.int32)
    return pooled, batch_offset
